# baseline (speedup 1.0000x reference)
import functools

import jax
import jax.numpy as jnp
from jax import lax
from jax.experimental import pallas as pl
from jax.experimental.pallas import tpu as pltpu
from jax.experimental.pallas import tpu_sc as plsc

N = 10000
E = 320000
D_IN = 128
DH = 16
NC = 2
NS = 16
NW = NC * NS
CHUNK = 128
CH_T = 80
EP = NW * CH_T * CHUNK
NACC = 10112
ZR = NACC // NS
NB = 8
HW = 16
PR = NACC // 8
CA = 80
CB = 80
CHMAX = max(CA, CB)

_MESH = plsc.VectorSubcoreMesh(core_axis_name="c", subcore_axis_name="s")
_SC_PARAMS = pltpu.CompilerParams(use_tc_tiling_on_sc=False)


HCH = E // CHUNK
HA = 78
HB0 = 79
HB1 = 78


def _sc_hist(dst2d):
    @functools.partial(
        pl.kernel,
        out_type=jax.ShapeDtypeStruct((NC * NACC, HW), jnp.float32),
        mesh=_MESH,
        compiler_params=_SC_PARAMS,
        scratch_types=[
            pltpu.VMEM((CHMAX, CHUNK), jnp.int32),
            pltpu.VMEM((CHUNK, HW), jnp.float32),
            pltpu.VMEM((ZR, HW), jnp.float32),
            pltpu.VMEM_SHARED((NACC, HW), jnp.float32),
            pltpu.SemaphoreType.DMA,
        ],
    )
    def k(dst_hbm, out_hbm, dstv, ones_v, zbuf, acc, hsem):
        cid = lax.axis_index("c")
        sid = lax.axis_index("s")

        @pl.loop(0, ZR)
        def _(i):
            zbuf.at[pl.ds(i, 1), pl.ds(0, HW)][...] = jnp.zeros((1, HW), jnp.float32)

        @pl.loop(0, CHUNK)
        def _(i):
            ones_v.at[pl.ds(i, 1), pl.ds(0, HW)][...] = jnp.ones((1, HW), jnp.float32)

        pltpu.sync_copy(zbuf, acc.at[pl.ds(sid * ZR, ZR)])
        plsc.subcore_barrier()

        def run(cnt, base):
            pltpu.sync_copy(dst_hbm.at[pl.ds(base, cnt)], dstv.at[pl.ds(0, cnt)])

            @pl.loop(0, cnt)
            def _(j):
                pltpu.async_copy(ones_v, acc.at[dstv.at[j]], hsem, add=True)

            @pl.loop(0, cnt)
            def _(j):
                pltpu.make_async_copy(ones_v, acc.at[dstv.at[j]], hsem).wait()

        @pl.when(cid == 0)
        def _():
            run(HA, sid * HA)

        @pl.when(jnp.logical_and(cid == 1, sid < 4))
        def _():
            run(HB0, NS * HA + sid * HB0)

        @pl.when(jnp.logical_and(cid == 1, sid >= 4))
        def _():
            run(HB1, NS * HA + 4 * HB0 + (sid - 4) * HB1)

        plsc.subcore_barrier()
        pltpu.sync_copy(
            acc.at[pl.ds(sid * ZR, ZR)],
            out_hbm.at[pl.ds(cid * NACC + sid * ZR, ZR)],
        )

    return k(dst2d)


def _sc_agg(src2d, dst2d, y):
    @functools.partial(
        pl.kernel,
        out_type=jax.ShapeDtypeStruct((NC * NACC, DH), jnp.float32),
        mesh=_MESH,
        compiler_params=_SC_PARAMS,
        scratch_types=[
            pltpu.VMEM((CHMAX, CHUNK), jnp.int32),
            pltpu.VMEM((CHMAX, CHUNK), jnp.int32),
            [pltpu.VMEM((CHUNK, DH), jnp.float32)] * NB,
            pltpu.VMEM((ZR, DH), jnp.float32),
            pltpu.VMEM_SHARED((NACC, DH), jnp.float32),
            pltpu.VMEM_SHARED((NACC, DH), jnp.float32),
            [pltpu.SemaphoreType.DMA] * NB,
            [pltpu.SemaphoreType.DMA] * NB,
        ],
    )
    def k(src_hbm, dst_hbm, y_hbm, out_hbm, srcv, dstv, rows, zbuf, acc, ysh, gs, ss):
        cid = lax.axis_index("c")
        sid = lax.axis_index("s")

        @pl.loop(0, ZR)
        def _(i):
            zbuf.at[pl.ds(i, 1), pl.ds(0, DH)][...] = jnp.zeros((1, DH), jnp.float32)

        pltpu.sync_copy(zbuf, acc.at[pl.ds(sid * ZR, ZR)])
        pltpu.sync_copy(y_hbm.at[pl.ds(sid * ZR, ZR)], ysh.at[pl.ds(sid * ZR, ZR)])
        plsc.subcore_barrier()

        def run(cnt, base):
            pltpu.sync_copy(src_hbm.at[pl.ds(base, cnt)], srcv.at[pl.ds(0, cnt)])
            pltpu.sync_copy(dst_hbm.at[pl.ds(base, cnt)], dstv.at[pl.ds(0, cnt)])

            for b in range(NB):
                pltpu.async_copy(ysh.at[srcv.at[b]], rows[b], gs[b])

            @pl.loop(0, cnt - NB, step=NB)
            def _(j):
                for b in range(NB):
                    pltpu.make_async_copy(ysh.at[srcv.at[j + b]], rows[b], gs[b]).wait()
                    pltpu.async_copy(rows[b], acc.at[dstv.at[j + b]], ss[b], add=True)
                for b in range(NB):
                    pltpu.make_async_copy(rows[b], acc.at[dstv.at[j + b]], ss[b]).wait()
                    pltpu.async_copy(ysh.at[srcv.at[j + NB + b]], rows[b], gs[b])

            j0 = cnt - NB
            for b in range(NB):
                pltpu.make_async_copy(ysh.at[srcv.at[j0 + b]], rows[b], gs[b]).wait()
                pltpu.async_copy(rows[b], acc.at[dstv.at[j0 + b]], ss[b], add=True)
            for b in range(NB):
                pltpu.make_async_copy(rows[b], acc.at[dstv.at[j0 + b]], ss[b]).wait()

        @pl.when(cid == 0)
        def _():
            run(CA, sid * CA)

        @pl.when(cid == 1)
        def _():
            run(CB, NS * CA + sid * CB)

        plsc.subcore_barrier()
        pltpu.sync_copy(
            acc.at[pl.ds(sid * ZR, ZR)],
            out_hbm.at[pl.ds(cid * NACC + sid * ZR, ZR)],
        )

    return k(src2d, dst2d, y)


def _tc_xw(xp, W1):
    def body(x_ref, w_ref, o_ref):
        x3 = x_ref[...]
        w = w_ref[...]
        o_ref[...] = jnp.concatenate(
            [jnp.dot(x3[:, j, :], w, preferred_element_type=jnp.float32)
             for j in range(8)], axis=1)

    return pl.pallas_call(
        body,
        out_shape=jax.ShapeDtypeStruct((PR, 128), jnp.float32),
    )(xp.reshape(PR, 8, D_IN), W1)


def _tc_scale(histp, xw):
    def body(h_ref, x_ref, y_ref, d_ref):
        hp = h_ref[...]
        cnt = hp[:PR] + hp[PR:]
        dinv = lax.rsqrt(cnt + 1.0)
        y_ref[...] = x_ref[...] * dinv
        d_ref[...] = dinv

    return pl.pallas_call(
        body,
        out_shape=[
            jax.ShapeDtypeStruct((PR, 128), jnp.float32),
            jax.ShapeDtypeStruct((PR, 128), jnp.float32),
        ],
    )(histp, xw)


def _blockdiag(w):
    wt = jnp.tile(w, (8, 8))
    i = lax.broadcasted_iota(jnp.int32, (128, 128), 0) // DH
    j = lax.broadcasted_iota(jnp.int32, (128, 128), 1) // DH
    return jnp.where(i == j, wt, 0.0)


def _tc_mid(aggp, y1, dinv, W2, b1):
    def body(a_ref, y_ref, d_ref, w_ref, b_ref, o_ref):
        ap = a_ref[...]
        d = d_ref[...]
        z = (ap[:PR] + ap[PR:] + y_ref[...]) * d + jnp.tile(b_ref[...], (1, 8))
        h = jnp.maximum(z, 0.0)
        o_ref[...] = jnp.dot(h, _blockdiag(w_ref[...]),
                             preferred_element_type=jnp.float32) * d

    return pl.pallas_call(
        body,
        out_shape=jax.ShapeDtypeStruct((PR, 128), jnp.float32),
    )(aggp, y1, dinv, W2, b1)


def _tc_last(aggp, y2, dinv, b2):
    def body(a_ref, y_ref, d_ref, b_ref, o_ref):
        ap = a_ref[...]
        zp = (ap[:PR] + ap[PR:] + y_ref[...]) * d_ref[...] + jnp.tile(b_ref[...], (1, 8))
        m = jnp.max(zp, axis=1, keepdims=True)
        sp = zp - m
        e = jnp.exp(sp)
        i = lax.broadcasted_iota(jnp.int32, (128, 128), 0) // DH
        j = lax.broadcasted_iota(jnp.int32, (128, 128), 1) // DH
        ones_blk = jnp.where(i == j, 1.0, 0.0)
        gsum = jnp.dot(e, ones_blk, preferred_element_type=jnp.float32)
        op = sp - jnp.log(gsum)
        pieces = [op[:, DH * j:DH * (j + 1)].reshape(PR, 1, DH) for j in range(8)]
        o_ref[...] = jnp.concatenate(pieces, axis=1).reshape(NACC, DH)[:N]

    return pl.pallas_call(
        body,
        out_shape=jax.ShapeDtypeStruct((N, DH), jnp.float32),
    )(aggp, y2, dinv, b2)


def kernel(x, edge_index, W1, b1, W2, b2):
    src = edge_index[0]
    dst = edge_index[1]
    pad = jnp.full((EP - E,), N, jnp.int32)
    src2d = jnp.concatenate([src, pad]).reshape(NW * CH_T, CHUNK)
    dst2d = jnp.concatenate([dst, pad]).reshape(NW * CH_T, CHUNK)
    xp = jnp.zeros((NACC, D_IN), jnp.float32).at[:N].set(x)

    dst_raw = dst.reshape(HCH, CHUNK)
    histp = _sc_hist(dst_raw).reshape(NC * PR, 128)
    xw = _tc_xw(xp, W1)
    y1, dinv = _tc_scale(histp, xw)
    agg1 = _sc_agg(src2d, dst2d, y1.reshape(NACC, DH)).reshape(NC * PR, 128)
    y2 = _tc_mid(agg1, y1, dinv, W2, b1.reshape(1, DH))
    agg2 = _sc_agg(src2d, dst2d, y2.reshape(NACC, DH)).reshape(NC * PR, 128)
    out = _tc_last(agg2, y2, dinv, b2.reshape(1, DH))
    return out


# speedup vs baseline: 1.0477x; 1.0477x over previous
import functools

import jax
import jax.numpy as jnp
from jax import lax
from jax.experimental import pallas as pl
from jax.experimental.pallas import tpu as pltpu
from jax.experimental.pallas import tpu_sc as plsc

N = 10000
E = 320000
D_IN = 128
DH = 16
NC = 2
NS = 16
NW = NC * NS
CHUNK = 128
CH_T = 80
EP = NW * CH_T * CHUNK
NACC = 10112
ZR = NACC // NS
NB = 8
HW = 16
PR = NACC // 8
CA = 80
CB = 80
CHMAX = max(CA, CB)

_MESH = plsc.VectorSubcoreMesh(core_axis_name="c", subcore_axis_name="s")
_SC_PARAMS = pltpu.CompilerParams(use_tc_tiling_on_sc=False)


HCH = E // CHUNK
HA = 78
HB0 = 79
HB1 = 78


def _sc_hist(dst2d):
    @functools.partial(
        pl.kernel,
        out_type=jax.ShapeDtypeStruct((NC * NACC, HW), jnp.float32),
        mesh=_MESH,
        compiler_params=_SC_PARAMS,
        scratch_types=[
            pltpu.VMEM((CHMAX, CHUNK), jnp.int32),
            pltpu.VMEM((CHUNK, HW), jnp.float32),
            pltpu.VMEM((ZR, HW), jnp.float32),
            pltpu.VMEM_SHARED((NACC, HW), jnp.float32),
            pltpu.SemaphoreType.DMA,
        ],
    )
    def k(dst_hbm, out_hbm, dstv, ones_v, zbuf, acc, hsem):
        cid = lax.axis_index("c")
        sid = lax.axis_index("s")

        @pl.loop(0, ZR)
        def _(i):
            zbuf.at[pl.ds(i, 1), pl.ds(0, HW)][...] = jnp.zeros((1, HW), jnp.float32)

        @pl.loop(0, CHUNK)
        def _(i):
            ones_v.at[pl.ds(i, 1), pl.ds(0, HW)][...] = jnp.ones((1, HW), jnp.float32)

        pltpu.sync_copy(zbuf, acc.at[pl.ds(sid * ZR, ZR)])
        plsc.subcore_barrier()

        def run(cnt, base):
            pltpu.sync_copy(dst_hbm.at[1, pl.ds(base, cnt)], dstv.at[pl.ds(0, cnt)])

            @pl.loop(0, cnt)
            def _(j):
                pltpu.async_copy(ones_v, acc.at[dstv.at[j]], hsem, add=True)

            @pl.loop(0, cnt)
            def _(j):
                pltpu.make_async_copy(ones_v, acc.at[dstv.at[j]], hsem).wait()

        @pl.when(cid == 0)
        def _():
            run(HA, sid * HA)

        @pl.when(jnp.logical_and(cid == 1, sid < 4))
        def _():
            run(HB0, NS * HA + sid * HB0)

        @pl.when(jnp.logical_and(cid == 1, sid >= 4))
        def _():
            run(HB1, NS * HA + 4 * HB0 + (sid - 4) * HB1)

        plsc.subcore_barrier()
        pltpu.sync_copy(
            acc.at[pl.ds(sid * ZR, ZR)],
            out_hbm.at[pl.ds(cid * NACC + sid * ZR, ZR)],
        )

    return k(dst2d)


def _sc_agg(src2d, dst2d, y):
    @functools.partial(
        pl.kernel,
        out_type=jax.ShapeDtypeStruct((NC * NACC, DH), jnp.float32),
        mesh=_MESH,
        compiler_params=_SC_PARAMS,
        scratch_types=[
            pltpu.VMEM((CHMAX, CHUNK), jnp.int32),
            pltpu.VMEM((CHMAX, CHUNK), jnp.int32),
            [pltpu.VMEM((CHUNK, DH), jnp.float32)] * NB,
            pltpu.VMEM((ZR, DH), jnp.float32),
            pltpu.VMEM_SHARED((NACC, DH), jnp.float32),
            pltpu.VMEM_SHARED((NACC, DH), jnp.float32),
            [pltpu.SemaphoreType.DMA] * NB,
            [pltpu.SemaphoreType.DMA] * NB,
        ],
    )
    def k(src_hbm, dst_hbm, y_hbm, out_hbm, srcv, dstv, rows, zbuf, acc, ysh, gs, ss):
        cid = lax.axis_index("c")
        sid = lax.axis_index("s")

        @pl.loop(0, ZR)
        def _(i):
            zbuf.at[pl.ds(i, 1), pl.ds(0, DH)][...] = jnp.zeros((1, DH), jnp.float32)

        pltpu.sync_copy(zbuf, acc.at[pl.ds(sid * ZR, ZR)])
        pltpu.sync_copy(y_hbm.at[pl.ds(sid * ZR, ZR)], ysh.at[pl.ds(sid * ZR, ZR)])
        plsc.subcore_barrier()

        def run(cnt, base):
            pltpu.sync_copy(src_hbm.at[pl.ds(base, cnt)], srcv.at[pl.ds(0, cnt)])
            pltpu.sync_copy(dst_hbm.at[pl.ds(base, cnt)], dstv.at[pl.ds(0, cnt)])

            for b in range(NB):
                pltpu.async_copy(ysh.at[srcv.at[b]], rows[b], gs[b])

            @pl.loop(0, cnt - NB, step=NB)
            def _(j):
                for b in range(NB):
                    pltpu.make_async_copy(ysh.at[srcv.at[j + b]], rows[b], gs[b]).wait()
                    pltpu.async_copy(rows[b], acc.at[dstv.at[j + b]], ss[b], add=True)
                for b in range(NB):
                    pltpu.make_async_copy(rows[b], acc.at[dstv.at[j + b]], ss[b]).wait()
                    pltpu.async_copy(ysh.at[srcv.at[j + NB + b]], rows[b], gs[b])

            j0 = cnt - NB
            for b in range(NB):
                pltpu.make_async_copy(ysh.at[srcv.at[j0 + b]], rows[b], gs[b]).wait()
                pltpu.async_copy(rows[b], acc.at[dstv.at[j0 + b]], ss[b], add=True)
            for b in range(NB):
                pltpu.make_async_copy(rows[b], acc.at[dstv.at[j0 + b]], ss[b]).wait()

        @pl.when(cid == 0)
        def _():
            run(CA, sid * CA)

        @pl.when(cid == 1)
        def _():
            run(CB, NS * CA + sid * CB)

        plsc.subcore_barrier()
        pltpu.sync_copy(
            acc.at[pl.ds(sid * ZR, ZR)],
            out_hbm.at[pl.ds(cid * NACC + sid * ZR, ZR)],
        )

    return k(src2d, dst2d, y)


def _tc_xw(xp, W1):
    def body(x_ref, w_ref, o_ref):
        x3 = x_ref[...]
        w = w_ref[...]
        o_ref[...] = jnp.concatenate(
            [jnp.dot(x3[:, j, :], w, preferred_element_type=jnp.float32)
             for j in range(8)], axis=1)

    return pl.pallas_call(
        body,
        out_shape=jax.ShapeDtypeStruct((PR, 128), jnp.float32),
    )(xp.reshape(PR, 8, D_IN), W1)


def _tc_scale(histp, xw):
    def body(h_ref, x_ref, y_ref, d_ref):
        hp = h_ref[...]
        cnt = hp[:PR] + hp[PR:]
        dinv = lax.rsqrt(cnt + 1.0)
        y_ref[...] = x_ref[...] * dinv
        d_ref[...] = dinv

    return pl.pallas_call(
        body,
        out_shape=[
            jax.ShapeDtypeStruct((PR, 128), jnp.float32),
            jax.ShapeDtypeStruct((PR, 128), jnp.float32),
        ],
    )(histp, xw)


def _blockdiag(w):
    wt = jnp.tile(w, (8, 8))
    i = lax.broadcasted_iota(jnp.int32, (128, 128), 0) // DH
    j = lax.broadcasted_iota(jnp.int32, (128, 128), 1) // DH
    return jnp.where(i == j, wt, 0.0)


def _tc_mid(aggp, y1, dinv, W2, b1):
    def body(a_ref, y_ref, d_ref, w_ref, b_ref, o_ref):
        ap = a_ref[...]
        d = d_ref[...]
        z = (ap[:PR] + ap[PR:] + y_ref[...]) * d + jnp.tile(b_ref[...], (1, 8))
        h = jnp.maximum(z, 0.0)
        o_ref[...] = jnp.dot(h, _blockdiag(w_ref[...]),
                             preferred_element_type=jnp.float32) * d

    return pl.pallas_call(
        body,
        out_shape=jax.ShapeDtypeStruct((PR, 128), jnp.float32),
    )(aggp, y1, dinv, W2, b1)


def _tc_last(aggp, y2, dinv, b2):
    def body(a_ref, y_ref, d_ref, b_ref, o_ref):
        ap = a_ref[...]
        zp = (ap[:PR] + ap[PR:] + y_ref[...]) * d_ref[...] + jnp.tile(b_ref[...], (1, 8))
        m = jnp.max(zp, axis=1, keepdims=True)
        sp = zp - m
        e = jnp.exp(sp)
        i = lax.broadcasted_iota(jnp.int32, (128, 128), 0) // DH
        j = lax.broadcasted_iota(jnp.int32, (128, 128), 1) // DH
        ones_blk = jnp.where(i == j, 1.0, 0.0)
        gsum = jnp.dot(e, ones_blk, preferred_element_type=jnp.float32)
        o_ref[...] = sp - jnp.log(gsum)

    return pl.pallas_call(
        body,
        out_shape=jax.ShapeDtypeStruct((PR, 128), jnp.float32),
    )(aggp, y2, dinv, b2)


def kernel(x, edge_index, W1, b1, W2, b2):
    src = edge_index[0]
    dst = edge_index[1]
    pad = jnp.full((EP - E,), N, jnp.int32)
    src2d = jnp.concatenate([src, pad]).reshape(NW * CH_T, CHUNK)
    dst2d = jnp.concatenate([dst, pad]).reshape(NW * CH_T, CHUNK)
    xp = jnp.zeros((NACC, D_IN), jnp.float32).at[:N].set(x)

    histp = _sc_hist(edge_index.reshape(2, HCH, CHUNK)).reshape(NC * PR, 128)
    xw = _tc_xw(xp, W1)
    y1, dinv = _tc_scale(histp, xw)
    agg1 = _sc_agg(src2d, dst2d, y1.reshape(NACC, DH)).reshape(NC * PR, 128)
    y2 = _tc_mid(agg1, y1, dinv, W2, b1.reshape(1, DH))
    agg2 = _sc_agg(src2d, dst2d, y2.reshape(NACC, DH)).reshape(NC * PR, 128)
    out = _tc_last(agg2, y2, dinv, b2.reshape(1, DH))
    return out.reshape(NACC, DH)[:N]
